# baseline (speedup 1.0000x reference)
import jax
import jax.numpy as jnp
from jax.experimental import pallas as pl
from jax.experimental.pallas import tpu as pltpu

_BM = 1024
_NSPLIT = 4


def _gate_kernel(*refs):
    w_ref = refs[_NSPLIT]
    out_ref = refs[_NSPLIT + 1]
    w = w_ref[...]
    for s in range(_NSPLIT):
        out_ref[s] = jax.lax.dot_general(
            refs[s][0], w,
            dimension_numbers=(((1,), (1,)), ((), ())),
            preferred_element_type=jnp.float32,
        )


def kernel(x, W):
    m, d = x.shape
    e = W.shape[0]
    xr = x.reshape(_NSPLIT, m // _NSPLIT, d)
    steps = m // _NSPLIT // _BM
    in_specs = [
        pl.BlockSpec((1, _BM, d), lambda i, s=s: (s, i, 0))
        for s in range(_NSPLIT)
    ]
    in_specs.append(pl.BlockSpec((e, d), lambda i: (0, 0)))
    logits = pl.pallas_call(
        _gate_kernel,
        grid=(steps,),
        in_specs=in_specs,
        out_specs=pl.BlockSpec((_NSPLIT, _BM, e), lambda i: (0, i, 0)),
        out_shape=jax.ShapeDtypeStruct((_NSPLIT, m // _NSPLIT, e), jnp.float32),
        compiler_params=pltpu.CompilerParams(
            dimension_semantics=("arbitrary",),
        ),
    )(*([xr] * _NSPLIT), W)
    return (logits.reshape(m, e), 2)


# speedup vs baseline: 1.2447x; 1.2447x over previous
import jax
import jax.numpy as jnp
from jax.experimental import pallas as pl
from jax.experimental.pallas import tpu as pltpu

_BM = 4096


def _gate_kernel(x_ref, w_ref, out_ref):
    out_ref[...] = jax.lax.dot_general(
        x_ref[...], w_ref[...],
        dimension_numbers=(((1,), (1,)), ((), ())),
        preferred_element_type=jnp.float32,
    )


def kernel(x, W):
    m, d = x.shape
    e = W.shape[0]
    grid = (m // _BM,)
    logits = pl.pallas_call(
        _gate_kernel,
        grid=grid,
        in_specs=[
            pl.BlockSpec((_BM, d), lambda i: (i, 0)),
            pl.BlockSpec((e, d), lambda i: (0, 0)),
        ],
        out_specs=pl.BlockSpec((_BM, e), lambda i: (i, 0)),
        out_shape=jax.ShapeDtypeStruct((m, e), jnp.float32),
        compiler_params=pltpu.CompilerParams(
            dimension_semantics=("parallel",),
        ),
    )(x, W)
    return (logits, 2)


# speedup vs baseline: 1.7272x; 1.3876x over previous
import jax
import jax.numpy as jnp
from jax.experimental import pallas as pl
from jax.experimental.pallas import tpu as pltpu

_BM = 4096


def _gate_kernel(x_ref, w_ref, out_ref):
    out_ref[...] = jax.lax.dot_general(
        w_ref[...], x_ref[...],
        dimension_numbers=(((1,), (1,)), ((), ())),
        preferred_element_type=jnp.float32,
    )


def kernel(x, W):
    m, d = x.shape
    e = W.shape[0]
    grid = (m // _BM,)
    logits_t = pl.pallas_call(
        _gate_kernel,
        grid=grid,
        in_specs=[
            pl.BlockSpec((_BM, d), lambda i: (i, 0)),
            pl.BlockSpec((e, d), lambda i: (0, 0)),
        ],
        out_specs=pl.BlockSpec((e, _BM), lambda i: (0, i)),
        out_shape=jax.ShapeDtypeStruct((e, m), jnp.float32),
        compiler_params=pltpu.CompilerParams(
            dimension_semantics=("parallel",),
        ),
    )(x, W)
    return (logits_t.T, 2)
